# baseline (speedup 1.0000x reference)
import functools

import jax
import jax.numpy as jnp
from jax import lax
from jax.experimental import pallas as pl
from jax.experimental.pallas import tpu as pltpu
from jax.experimental.pallas import tpu_sc as plsc

B = 16384
CTX = 20
D = 64
L = 16
NC = 2
NS = 16
NW = NC * NS
ROWS_PER_W = B // NW
CHUNK = 32
N_CHUNKS = ROWS_PER_W // CHUNK
IDX_PER_CHUNK = CHUNK * CTX
G = 128
NG = IDX_PER_CHUNK // G


def _make_kernel():
    mesh = plsc.VectorSubcoreMesh(
        core_axis_name="c", subcore_axis_name="s", num_cores=NC, num_subcores=NS
    )

    @functools.partial(
        pl.kernel,
        out_type=jax.ShapeDtypeStruct((B, D), jnp.float32),
        mesh=mesh,
        compiler_params=pltpu.CompilerParams(use_tc_tiling_on_sc=False),
        scratch_types=[
            pltpu.VMEM((IDX_PER_CHUNK,), jnp.int32),
            pltpu.VMEM((IDX_PER_CHUNK, D), jnp.float32),
            pltpu.VMEM((CHUNK, D), jnp.float32),
            pltpu.SemaphoreType.DMA,
        ],
    )
    def cbow_kernel(x_hbm, table_hbm, out_hbm, idx_v, rows_v, out_v, sem):
        wid = lax.axis_index("s") * NC + lax.axis_index("c")
        base = wid * ROWS_PER_W

        def chunk_body(ci, carry):
            cbase = base + ci * CHUNK
            pltpu.sync_copy(x_hbm.at[pl.ds(cbase * CTX, IDX_PER_CHUNK)], idx_v)
            descs = [
                pltpu.async_copy(
                    table_hbm.at[idx_v.at[pl.ds(g * G, G)]],
                    rows_v.at[pl.ds(g * G, G)],
                    sem,
                )
                for g in range(NG)
            ]
            for d in descs:
                d.wait()

            def red_body(b, carry2):
                r0 = b * CTX
                for k in range(D // L):
                    acc = rows_v[r0, pl.ds(k * L, L)]
                    for j in range(1, CTX):
                        acc = acc + rows_v[r0 + j, pl.ds(k * L, L)]
                    out_v[b, pl.ds(k * L, L)] = acc * jnp.float32(1.0 / CTX)
                return carry2

            lax.fori_loop(0, CHUNK, red_body, 0)
            pltpu.sync_copy(out_v, out_hbm.at[pl.ds(cbase, CHUNK)])
            return carry

        lax.fori_loop(0, N_CHUNKS, chunk_body, 0)

    return cbow_kernel


_cbow = _make_kernel()


@jax.jit
def kernel(x, y, table):
    del y
    x_flat = x.astype(jnp.int32).reshape(B * CTX)
    return _cbow(x_flat, table)
